# baseline (speedup 1.0000x reference)
import functools

import jax
import jax.numpy as jnp
from jax import lax
from jax.experimental import pallas as pl
from jax.experimental.pallas import tpu as pltpu
from jax.experimental.pallas import tpu_sc as plsc

N = 100000
E = 1600000
FEAT = 16
D = 64

NC = 2
NT = 16
LANES = 16

CHUNK = 25088
NDUMP = 256
ACC = CHUNK + NDUMP
NPAD = 4 * CHUNK

BLK = 1024
NBLK = 98
NROWS = NBLK * BLK

EB = 64
E_PAD = ((E + NT * EB - 1) // (NT * EB)) * (NT * EB)
EPT = E_PAD // NT
NB = EPT // EB
E_EXT = E_PAD + 5 * EB
EPT2 = E_PAD // (NT * NC)
NB2 = EPT2 // EB

_mesh = plsc.VectorSubcoreMesh(core_axis_name="c", subcore_axis_name="s",
                               num_cores=NC, num_subcores=NT)


@functools.partial(
    pl.kernel,
    out_type=jax.ShapeDtypeStruct((NC, NROWS), jnp.float32),
    mesh=_mesh,
    scratch_types=[
        pltpu.VMEM((4, EB), jnp.int32),
        pltpu.VMEM((EB,), jnp.float32),
        pltpu.VMEM_SHARED((NROWS,), jnp.float32),
        pltpu.SemaphoreType.DMA,
        pltpu.SemaphoreType.DMA,
    ],
)
def _deg_kernel(col_h, zeros_h, degp_h, col_v, ones_v, acc_sh,
                sem_c, sem_s):
    c = lax.axis_index("c")
    s = lax.axis_index("s")
    for k in range(EB // LANES):
        ones_v[pl.ds(k * LANES, LANES)] = jnp.ones((LANES,), jnp.float32)
    DS = NROWS // NT
    pltpu.sync_copy(zeros_h.at[pl.ds(s * DS, DS)], acc_sh.at[pl.ds(s * DS, DS)])
    plsc.subcore_barrier()
    ebase = (c * NT + s) * EPT2

    def fire_col(b, slot):
        pltpu.async_copy(col_h.at[pl.ds(ebase + b * EB, EB)], col_v.at[slot],
                         sem_c)

    def wait_col():
        pltpu.make_async_copy(col_h.at[pl.ds(0, EB)], col_v.at[0], sem_c).wait()

    def wait_sc():
        pltpu.make_async_copy(ones_v, acc_sh.at[col_v.at[0]], sem_s).wait()

    for b in range(2):
        fire_col(b, b)

    def body(b, carry):
        fire_col(b + 2, lax.rem(b + 2, 4))
        wait_col()

        @pl.when(b >= 2)
        def _():
            wait_sc()

        pltpu.async_copy(ones_v, acc_sh.at[col_v.at[lax.rem(b, 4)]], sem_s,
                         add=True)
        return carry

    lax.fori_loop(0, NB2, body, 0)
    for _ in range(2):
        wait_sc()
        wait_col()
    plsc.subcore_barrier()
    pltpu.sync_copy(acc_sh.at[pl.ds(s * DS, DS)], degp_h.at[c, pl.ds(s * DS, DS)])


NS_R = 6
NS_I = 4
IDX_AHEAD = 3
GATH_AHEAD = 2
SC_BEHIND = 4


@functools.partial(
    pl.kernel,
    out_type=jax.ShapeDtypeStruct((NPAD, D), jnp.float32),
    mesh=_mesh,
    compiler_params=pltpu.CompilerParams(use_tc_tiling_on_sc=False),
    scratch_types=[
        pltpu.VMEM((NS_I, EB), jnp.int32),
        pltpu.VMEM((NS_I, EB), jnp.int32),
        pltpu.VMEM((NS_I, EB), jnp.int32),
        pltpu.VMEM((NS_R, EB, D), jnp.float32),
        pltpu.VMEM_SHARED((ACC, D), jnp.float32),
        pltpu.SemaphoreType.DMA,
        pltpu.SemaphoreType.DMA,
        pltpu.SemaphoreType.DMA,
        pltpu.SemaphoreType.DMA,
    ],
)
def _scatter_kernel(g_h, row_h, col_h, zeros_h, out_h,
                    row_v, col_v, lcol_v, rows_v, acc_sh,
                    sem_r, sem_c, sem_g, sem_s):
    c = lax.axis_index("c")
    s = lax.axis_index("s")
    ebase = s * EPT
    ZS = ACC // NT
    WS = CHUNK // NT

    def fire_idx(b, slot):
        off = ebase + b * EB
        pltpu.async_copy(row_h.at[pl.ds(off, EB)], row_v.at[slot], sem_r)
        pltpu.async_copy(col_h.at[pl.ds(off, EB)], col_v.at[slot], sem_c)

    def wait_idx():
        pltpu.make_async_copy(row_h.at[pl.ds(0, EB)], row_v.at[0], sem_r).wait()
        pltpu.make_async_copy(col_h.at[pl.ds(0, EB)], col_v.at[0], sem_c).wait()

    def fire_gather(bslot, rslot):
        pltpu.async_copy(g_h.at[row_v.at[bslot]], rows_v.at[rslot], sem_g)

    def wait_gather():
        pltpu.make_async_copy(g_h.at[row_v.at[0]], rows_v.at[0], sem_g).wait()

    def fire_scatter(bslot, rslot):
        pltpu.async_copy(rows_v.at[rslot], acc_sh.at[lcol_v.at[bslot]], sem_s,
                         add=True)

    def wait_scatter():
        pltpu.make_async_copy(rows_v.at[0], acc_sh.at[lcol_v.at[0]],
                              sem_s).wait()

    for p in range(2):
        lo = (c * 2 + p) * CHUNK
        pltpu.sync_copy(zeros_h.at[pl.ds(s * ZS, ZS)], acc_sh.at[pl.ds(s * ZS, ZS)])
        plsc.subcore_barrier()

        for b in range(IDX_AHEAD):
            fire_idx(b, b)
        for b in range(GATH_AHEAD):
            wait_idx()
            fire_gather(b, b)

        def body(b, carry):
            fire_idx(b + IDX_AHEAD, lax.rem(b + IDX_AHEAD, NS_I))
            wait_idx()

            @pl.when(b >= SC_BEHIND)
            def _():
                wait_scatter()

            fire_gather(lax.rem(b + GATH_AHEAD, NS_I),
                        lax.rem(b + GATH_AHEAD, NS_R))
            kb = lax.rem(b, NS_I)
            for k in range(EB // LANES):
                cv = col_v[kb, pl.ds(k * LANES, LANES)]
                l = cv - lo
                valid = (l >= 0) & (l < CHUNK)
                dump = CHUNK + (cv & (NDUMP - 1))
                lcol_v[kb, pl.ds(k * LANES, LANES)] = jnp.where(valid, l, dump)
            wait_gather()
            fire_scatter(kb, lax.rem(b, NS_R))
            return carry

        lax.fori_loop(0, NB, body, 0)
        for _ in range(SC_BEHIND):
            wait_scatter()
        for _ in range(GATH_AHEAD):
            wait_gather()
        for _ in range(IDX_AHEAD - GATH_AHEAD):
            wait_idx()
        plsc.subcore_barrier()
        pltpu.sync_copy(acc_sh.at[pl.ds(s * WS, WS)],
                        out_h.at[pl.ds(lo + s * WS, WS)])
        plsc.subcore_barrier()


def _tc0_body(x_ref, degp_ref, We_ref, be_ref, W1_ref, h_ref, g_ref, dis_ref):
    deg = degp_ref[0] + degp_ref[1] + 1.0
    dis = lax.rsqrt(deg)
    h = jnp.dot(x_ref[...], We_ref[...], preferred_element_type=jnp.float32) + be_ref[...]
    g = jnp.dot(h, W1_ref[...], preferred_element_type=jnp.float32) * dis
    h_ref[...] = h
    g_ref[...] = g
    dis_ref[...] = dis


def _tc0(x_pad, degp3, W_emb, b_emb2, W1):
    return pl.pallas_call(
        _tc0_body,
        grid=(NBLK,),
        in_specs=[
            pl.BlockSpec((BLK, FEAT), lambda i: (i, 0)),
            pl.BlockSpec((NC, BLK, 1), lambda i: (0, i, 0)),
            pl.BlockSpec((FEAT, D), lambda i: (0, 0)),
            pl.BlockSpec((1, D), lambda i: (0, 0)),
            pl.BlockSpec((D, D), lambda i: (0, 0)),
        ],
        out_specs=[
            pl.BlockSpec((BLK, D), lambda i: (i, 0)),
            pl.BlockSpec((BLK, D), lambda i: (i, 0)),
            pl.BlockSpec((BLK, 1), lambda i: (i, 0)),
        ],
        out_shape=[
            jax.ShapeDtypeStruct((NROWS, D), jnp.float32),
            jax.ShapeDtypeStruct((NROWS, D), jnp.float32),
            jax.ShapeDtypeStruct((NROWS, 1), jnp.float32),
        ],
    )(x_pad, degp3, W_emb, b_emb2, W1)


def _mid_body(h_ref, g_ref, S_ref, dis_ref, b_ref, W_ref, hn_ref, gn_ref):
    dis = dis_ref[...]
    hn = h_ref[...] + jnp.maximum(dis * (S_ref[...] + g_ref[...]) + b_ref[...], 0.0)
    gn = jnp.dot(hn, W_ref[...], preferred_element_type=jnp.float32) * dis
    hn_ref[...] = hn
    gn_ref[...] = gn


def _mid(h, g, S, dis, b2, W_next):
    return pl.pallas_call(
        _mid_body,
        grid=(NBLK,),
        in_specs=[
            pl.BlockSpec((BLK, D), lambda i: (i, 0)),
            pl.BlockSpec((BLK, D), lambda i: (i, 0)),
            pl.BlockSpec((BLK, D), lambda i: (i, 0)),
            pl.BlockSpec((BLK, 1), lambda i: (i, 0)),
            pl.BlockSpec((1, D), lambda i: (0, 0)),
            pl.BlockSpec((D, D), lambda i: (0, 0)),
        ],
        out_specs=[
            pl.BlockSpec((BLK, D), lambda i: (i, 0)),
            pl.BlockSpec((BLK, D), lambda i: (i, 0)),
        ],
        out_shape=[
            jax.ShapeDtypeStruct((NROWS, D), jnp.float32),
            jax.ShapeDtypeStruct((NROWS, D), jnp.float32),
        ],
    )(h, g, S, dis, b2, W_next)


def _fin_body(h_ref, g_ref, S_ref, dis_ref, b_ref, Wn1_ref, bn1_ref, Wn2_ref,
              bn2_ref, Wg1_ref, bg1_ref, Wg2_ref, bg2_ref,
              h_out_ref, np_ref, gp_ref, hsum_ref):
    i = pl.program_id(0)
    h = h_ref[...] + jnp.maximum(
        dis_ref[...] * (S_ref[...] + g_ref[...]) + b_ref[...], 0.0)
    t = jnp.maximum(
        jnp.dot(h, Wn1_ref[...], preferred_element_type=jnp.float32) + bn1_ref[...], 0.0)
    np_ref[...] = jnp.dot(t, Wn2_ref[...], preferred_element_type=jnp.float32) + bn2_ref[...]
    h_out_ref[...] = h
    ridx = i * BLK + lax.broadcasted_iota(jnp.int32, (BLK, 1), 0)
    hm = jnp.where(ridx < N, h, 0.0)

    @pl.when(i == 0)
    def _():
        hsum_ref[...] = jnp.zeros_like(hsum_ref)

    hsum_ref[...] += jnp.sum(hm, axis=0, keepdims=True)

    @pl.when(i == NBLK - 1)
    def _():
        gm = hsum_ref[...] * (1.0 / N)
        tg = jnp.maximum(
            jnp.dot(gm, Wg1_ref[...], preferred_element_type=jnp.float32)
            + bg1_ref[...], 0.0)
        z = jnp.dot(tg, Wg2_ref[...], preferred_element_type=jnp.float32) + bg2_ref[...]
        gp_ref[...] = 1.0 / (1.0 + jnp.exp(-z))


def _fin(h, g, S, dis, b2, Wn1, bn1_2, Wn2, bn2_2, Wg1, bg1_2, Wg2, bg2_2):
    return pl.pallas_call(
        _fin_body,
        grid=(NBLK,),
        in_specs=[
            pl.BlockSpec((BLK, D), lambda i: (i, 0)),
            pl.BlockSpec((BLK, D), lambda i: (i, 0)),
            pl.BlockSpec((BLK, D), lambda i: (i, 0)),
            pl.BlockSpec((BLK, 1), lambda i: (i, 0)),
            pl.BlockSpec((1, D), lambda i: (0, 0)),
            pl.BlockSpec((D, D), lambda i: (0, 0)),
            pl.BlockSpec((1, D), lambda i: (0, 0)),
            pl.BlockSpec((D, 13), lambda i: (0, 0)),
            pl.BlockSpec((1, 13), lambda i: (0, 0)),
            pl.BlockSpec((D, D // 2), lambda i: (0, 0)),
            pl.BlockSpec((1, D // 2), lambda i: (0, 0)),
            pl.BlockSpec((D // 2, 1), lambda i: (0, 0)),
            pl.BlockSpec((1, 1), lambda i: (0, 0)),
        ],
        out_specs=[
            pl.BlockSpec((BLK, D), lambda i: (i, 0)),
            pl.BlockSpec((BLK, 13), lambda i: (i, 0)),
            pl.BlockSpec((1, 1), lambda i: (0, 0)),
            pl.BlockSpec((1, D), lambda i: (0, 0)),
        ],
        out_shape=[
            jax.ShapeDtypeStruct((NROWS, D), jnp.float32),
            jax.ShapeDtypeStruct((NROWS, 13), jnp.float32),
            jax.ShapeDtypeStruct((1, 1), jnp.float32),
            jax.ShapeDtypeStruct((1, D), jnp.float32),
        ],
    )(h, g, S, dis, b2, Wn1, bn1_2, Wn2, bn2_2, Wg1, bg1_2, Wg2, bg2_2)


def kernel(x, edge_index, W_emb, b_emb, W1, b1, W2, b2, W3, b3,
           Wn1, bn1, Wn2, bn2, Wg1, bg1, Wg2, bg2):
    padn = E_EXT - E
    j = jnp.arange(padn, dtype=jnp.int32)
    row_p = jnp.concatenate([edge_index[0], (j * 8191) % N])
    col_p = jnp.concatenate([edge_index[1], N + (j % 256)])

    x_pad = jnp.pad(x, ((0, NROWS - N), (0, 0)))
    zeros_acc = jnp.zeros((ACC, D), jnp.float32)
    zeros_deg = jnp.zeros((NROWS,), jnp.float32)

    degp = _deg_kernel(col_p, zeros_deg)
    degp3 = degp.reshape(NC, NROWS, 1)

    h0, g1, dis = _tc0(x_pad, degp3, W_emb, b_emb.reshape(1, D), W1)
    S1 = _scatter_kernel(g1, row_p, col_p, zeros_acc)
    h1, g2 = _mid(h0, g1, S1, dis, b1.reshape(1, D), W2)
    S2 = _scatter_kernel(g2, row_p, col_p, zeros_acc)
    h2, g3 = _mid(h1, g2, S2, dis, b2.reshape(1, D), W3)
    S3 = _scatter_kernel(g3, row_p, col_p, zeros_acc)
    h3, np_out, gp, _ = _fin(h2, g3, S3, dis, b3.reshape(1, D),
                             Wn1, bn1.reshape(1, D), Wn2, bn2.reshape(1, 13),
                             Wg1, bg1.reshape(1, D // 2), Wg2,
                             bg2.reshape(1, 1))

    return (h3[:N], np_out[:N, :10], np_out[:N, 10:13], gp)


# speedup vs baseline: 1.1217x; 1.1217x over previous
import functools

import jax
import jax.numpy as jnp
from jax import lax
from jax.experimental import pallas as pl
from jax.experimental.pallas import tpu as pltpu
from jax.experimental.pallas import tpu_sc as plsc

N = 100000
E = 1600000
FEAT = 16
D = 64

NC = 2
NT = 16
LANES = 16

CHUNK = 25088
NDUMP = 256
ACC = CHUNK + NDUMP
NPAD = 4 * CHUNK

BLK = 1024
NBLK = 98
NROWS = NBLK * BLK

EB = 128
E_PAD = ((E + NT * EB - 1) // (NT * EB)) * (NT * EB)
EPT = E_PAD // NT
NB = EPT // EB
E_EXT = E_PAD + 5 * EB
EPT2 = E_PAD // (NT * NC)
NB2 = EPT2 // EB

_mesh = plsc.VectorSubcoreMesh(core_axis_name="c", subcore_axis_name="s",
                               num_cores=NC, num_subcores=NT)


@functools.partial(
    pl.kernel,
    out_type=jax.ShapeDtypeStruct((NC, NROWS), jnp.float32),
    mesh=_mesh,
    scratch_types=[
        pltpu.VMEM((4, EB), jnp.int32),
        pltpu.VMEM((EB,), jnp.float32),
        pltpu.VMEM_SHARED((NROWS,), jnp.float32),
        pltpu.SemaphoreType.DMA,
        pltpu.SemaphoreType.DMA,
    ],
)
def _deg_kernel(rc_h, zeros_h, degp_h, col_v, ones_v, acc_sh,
                sem_c, sem_s):
    c = lax.axis_index("c")
    s = lax.axis_index("s")
    for k in range(EB // LANES):
        ones_v[pl.ds(k * LANES, LANES)] = jnp.ones((LANES,), jnp.float32)
    DS = NROWS // NT
    pltpu.sync_copy(zeros_h.at[pl.ds(s * DS, DS)], acc_sh.at[pl.ds(s * DS, DS)])
    plsc.subcore_barrier()
    bbase = (c * NT + s) * NB2

    def fire_col(b, slot):
        pltpu.async_copy(rc_h.at[pl.ds((bbase + b) * 2 * EB + EB, EB)],
                         col_v.at[slot], sem_c)

    def wait_col():
        pltpu.make_async_copy(rc_h.at[pl.ds(0, EB)], col_v.at[0], sem_c).wait()

    def wait_sc():
        pltpu.make_async_copy(ones_v, acc_sh.at[col_v.at[0]], sem_s).wait()

    for b in range(2):
        fire_col(b, b)

    def body(b, carry):
        fire_col(b + 2, lax.rem(b + 2, 4))
        wait_col()

        @pl.when(b >= 2)
        def _():
            wait_sc()

        pltpu.async_copy(ones_v, acc_sh.at[col_v.at[lax.rem(b, 4)]], sem_s,
                         add=True)
        return carry

    lax.fori_loop(0, NB2, body, 0)
    for _ in range(2):
        wait_sc()
        wait_col()
    plsc.subcore_barrier()
    pltpu.sync_copy(acc_sh.at[pl.ds(s * DS, DS)], degp_h.at[c, pl.ds(s * DS, DS)])


NS_R = 3
NS_I = 4
IDX_AHEAD = 2
GATH_AHEAD = 1
SC_BEHIND = 2


@functools.partial(
    pl.kernel,
    out_type=jax.ShapeDtypeStruct((NPAD, D), jnp.float32),
    mesh=_mesh,
    compiler_params=pltpu.CompilerParams(use_tc_tiling_on_sc=False),
    scratch_types=[
        pltpu.VMEM((NS_I, 2 * EB), jnp.int32),
        pltpu.VMEM((NS_I, EB), jnp.int32),
        pltpu.VMEM((NS_R, EB, D), jnp.float32),
        pltpu.VMEM_SHARED((ACC, D), jnp.float32),
        pltpu.SemaphoreType.DMA,
        pltpu.SemaphoreType.DMA,
        pltpu.SemaphoreType.DMA,
    ],
)
def _scatter_kernel(g_h, rc_h, zeros_h, out_h,
                    idx_v, lcol_v, rows_v, acc_sh,
                    sem_r, sem_g, sem_s):
    c = lax.axis_index("c")
    s = lax.axis_index("s")
    bbase = s * NB
    ZS = ACC // NT
    WS = CHUNK // NT

    def fire_idx(b, slot):
        pltpu.async_copy(rc_h.at[pl.ds((bbase + b) * 2 * EB, 2 * EB)],
                         idx_v.at[slot], sem_r)

    def wait_idx():
        pltpu.make_async_copy(rc_h.at[pl.ds(0, 2 * EB)], idx_v.at[0],
                              sem_r).wait()

    def fire_gather(bslot, rslot):
        pltpu.async_copy(g_h.at[idx_v.at[bslot, pl.ds(0, EB)]],
                         rows_v.at[rslot], sem_g)

    def wait_gather():
        pltpu.make_async_copy(g_h.at[idx_v.at[0, pl.ds(0, EB)]],
                              rows_v.at[0], sem_g).wait()

    def fire_scatter(bslot, rslot):
        pltpu.async_copy(rows_v.at[rslot], acc_sh.at[lcol_v.at[bslot]], sem_s,
                         add=True)

    def wait_scatter():
        pltpu.make_async_copy(rows_v.at[0], acc_sh.at[lcol_v.at[0]],
                              sem_s).wait()

    for p in range(2):
        lo = (c * 2 + p) * CHUNK
        pltpu.sync_copy(zeros_h.at[pl.ds(s * ZS, ZS)], acc_sh.at[pl.ds(s * ZS, ZS)])
        plsc.subcore_barrier()

        for b in range(IDX_AHEAD):
            fire_idx(b, b)
        for b in range(GATH_AHEAD):
            wait_idx()
            fire_gather(b, b)

        def body(b, carry):
            fire_idx(b + IDX_AHEAD, lax.rem(b + IDX_AHEAD, NS_I))
            wait_idx()

            @pl.when(b >= SC_BEHIND)
            def _():
                wait_scatter()

            fire_gather(lax.rem(b + GATH_AHEAD, NS_I),
                        lax.rem(b + GATH_AHEAD, NS_R))
            kb = lax.rem(b, NS_I)
            for k in range(EB // LANES):
                cv = idx_v[kb, pl.ds(EB + k * LANES, LANES)]
                l = cv - lo
                valid = (l >= 0) & (l < CHUNK)
                dump = CHUNK + (cv & (NDUMP - 1))
                lcol_v[kb, pl.ds(k * LANES, LANES)] = jnp.where(valid, l, dump)
            wait_gather()
            fire_scatter(kb, lax.rem(b, NS_R))
            return carry

        lax.fori_loop(0, NB, body, 0)
        for _ in range(SC_BEHIND):
            wait_scatter()
        for _ in range(GATH_AHEAD):
            wait_gather()
        for _ in range(IDX_AHEAD - GATH_AHEAD):
            wait_idx()
        plsc.subcore_barrier()
        pltpu.sync_copy(acc_sh.at[pl.ds(s * WS, WS)],
                        out_h.at[pl.ds(lo + s * WS, WS)])
        plsc.subcore_barrier()


def _tc0_body(x_ref, degp_ref, We_ref, be_ref, W1_ref, h_ref, g_ref, dis_ref):
    deg = degp_ref[0] + degp_ref[1] + 1.0
    dis = lax.rsqrt(deg)
    h = jnp.dot(x_ref[...], We_ref[...], preferred_element_type=jnp.float32) + be_ref[...]
    g = jnp.dot(h, W1_ref[...], preferred_element_type=jnp.float32) * dis
    h_ref[...] = h
    g_ref[...] = g
    dis_ref[...] = dis


def _tc0(x_pad, degp3, W_emb, b_emb2, W1):
    return pl.pallas_call(
        _tc0_body,
        grid=(NBLK,),
        in_specs=[
            pl.BlockSpec((BLK, FEAT), lambda i: (i, 0)),
            pl.BlockSpec((NC, BLK, 1), lambda i: (0, i, 0)),
            pl.BlockSpec((FEAT, D), lambda i: (0, 0)),
            pl.BlockSpec((1, D), lambda i: (0, 0)),
            pl.BlockSpec((D, D), lambda i: (0, 0)),
        ],
        out_specs=[
            pl.BlockSpec((BLK, D), lambda i: (i, 0)),
            pl.BlockSpec((BLK, D), lambda i: (i, 0)),
            pl.BlockSpec((BLK, 1), lambda i: (i, 0)),
        ],
        out_shape=[
            jax.ShapeDtypeStruct((NROWS, D), jnp.float32),
            jax.ShapeDtypeStruct((NROWS, D), jnp.float32),
            jax.ShapeDtypeStruct((NROWS, 1), jnp.float32),
        ],
    )(x_pad, degp3, W_emb, b_emb2, W1)


def _mid_body(h_ref, g_ref, S_ref, dis_ref, b_ref, W_ref, hn_ref, gn_ref):
    dis = dis_ref[...]
    hn = h_ref[...] + jnp.maximum(dis * (S_ref[...] + g_ref[...]) + b_ref[...], 0.0)
    gn = jnp.dot(hn, W_ref[...], preferred_element_type=jnp.float32) * dis
    hn_ref[...] = hn
    gn_ref[...] = gn


def _mid(h, g, S, dis, b2, W_next):
    return pl.pallas_call(
        _mid_body,
        grid=(NBLK,),
        in_specs=[
            pl.BlockSpec((BLK, D), lambda i: (i, 0)),
            pl.BlockSpec((BLK, D), lambda i: (i, 0)),
            pl.BlockSpec((BLK, D), lambda i: (i, 0)),
            pl.BlockSpec((BLK, 1), lambda i: (i, 0)),
            pl.BlockSpec((1, D), lambda i: (0, 0)),
            pl.BlockSpec((D, D), lambda i: (0, 0)),
        ],
        out_specs=[
            pl.BlockSpec((BLK, D), lambda i: (i, 0)),
            pl.BlockSpec((BLK, D), lambda i: (i, 0)),
        ],
        out_shape=[
            jax.ShapeDtypeStruct((NROWS, D), jnp.float32),
            jax.ShapeDtypeStruct((NROWS, D), jnp.float32),
        ],
    )(h, g, S, dis, b2, W_next)


def _fin_body(h_ref, g_ref, S_ref, dis_ref, b_ref, Wn1_ref, bn1_ref, Wn2_ref,
              bn2_ref, Wg1_ref, bg1_ref, Wg2_ref, bg2_ref,
              h_out_ref, np_ref, gp_ref, hsum_ref):
    i = pl.program_id(0)
    h = h_ref[...] + jnp.maximum(
        dis_ref[...] * (S_ref[...] + g_ref[...]) + b_ref[...], 0.0)
    t = jnp.maximum(
        jnp.dot(h, Wn1_ref[...], preferred_element_type=jnp.float32) + bn1_ref[...], 0.0)
    np_ref[...] = jnp.dot(t, Wn2_ref[...], preferred_element_type=jnp.float32) + bn2_ref[...]
    h_out_ref[...] = h
    ridx = i * BLK + lax.broadcasted_iota(jnp.int32, (BLK, 1), 0)
    hm = jnp.where(ridx < N, h, 0.0)

    @pl.when(i == 0)
    def _():
        hsum_ref[...] = jnp.zeros_like(hsum_ref)

    hsum_ref[...] += jnp.sum(hm, axis=0, keepdims=True)

    @pl.when(i == NBLK - 1)
    def _():
        gm = hsum_ref[...] * (1.0 / N)
        tg = jnp.maximum(
            jnp.dot(gm, Wg1_ref[...], preferred_element_type=jnp.float32)
            + bg1_ref[...], 0.0)
        z = jnp.dot(tg, Wg2_ref[...], preferred_element_type=jnp.float32) + bg2_ref[...]
        gp_ref[...] = 1.0 / (1.0 + jnp.exp(-z))


def _fin(h, g, S, dis, b2, Wn1, bn1_2, Wn2, bn2_2, Wg1, bg1_2, Wg2, bg2_2):
    return pl.pallas_call(
        _fin_body,
        grid=(NBLK,),
        in_specs=[
            pl.BlockSpec((BLK, D), lambda i: (i, 0)),
            pl.BlockSpec((BLK, D), lambda i: (i, 0)),
            pl.BlockSpec((BLK, D), lambda i: (i, 0)),
            pl.BlockSpec((BLK, 1), lambda i: (i, 0)),
            pl.BlockSpec((1, D), lambda i: (0, 0)),
            pl.BlockSpec((D, D), lambda i: (0, 0)),
            pl.BlockSpec((1, D), lambda i: (0, 0)),
            pl.BlockSpec((D, 13), lambda i: (0, 0)),
            pl.BlockSpec((1, 13), lambda i: (0, 0)),
            pl.BlockSpec((D, D // 2), lambda i: (0, 0)),
            pl.BlockSpec((1, D // 2), lambda i: (0, 0)),
            pl.BlockSpec((D // 2, 1), lambda i: (0, 0)),
            pl.BlockSpec((1, 1), lambda i: (0, 0)),
        ],
        out_specs=[
            pl.BlockSpec((BLK, D), lambda i: (i, 0)),
            pl.BlockSpec((BLK, 13), lambda i: (i, 0)),
            pl.BlockSpec((1, 1), lambda i: (0, 0)),
            pl.BlockSpec((1, D), lambda i: (0, 0)),
        ],
        out_shape=[
            jax.ShapeDtypeStruct((NROWS, D), jnp.float32),
            jax.ShapeDtypeStruct((NROWS, 13), jnp.float32),
            jax.ShapeDtypeStruct((1, 1), jnp.float32),
            jax.ShapeDtypeStruct((1, D), jnp.float32),
        ],
    )(h, g, S, dis, b2, Wn1, bn1_2, Wn2, bn2_2, Wg1, bg1_2, Wg2, bg2_2)


def kernel(x, edge_index, W_emb, b_emb, W1, b1, W2, b2, W3, b3,
           Wn1, bn1, Wn2, bn2, Wg1, bg1, Wg2, bg2):
    padn = E_EXT - E
    j = jnp.arange(padn, dtype=jnp.int32)
    row_p = jnp.concatenate([edge_index[0], (j * 8191) % N])
    col_p = jnp.concatenate([edge_index[1], N + (j % 256)])
    rc = jnp.stack([row_p.reshape(-1, EB), col_p.reshape(-1, EB)],
                   axis=1).reshape(-1)

    x_pad = jnp.pad(x, ((0, NROWS - N), (0, 0)))
    zeros_acc = jnp.zeros((ACC, D), jnp.float32)
    zeros_deg = jnp.zeros((NROWS,), jnp.float32)

    degp = _deg_kernel(rc, zeros_deg)
    degp3 = degp.reshape(NC, NROWS, 1)

    h0, g1, dis = _tc0(x_pad, degp3, W_emb, b_emb.reshape(1, D), W1)
    S1 = _scatter_kernel(g1, rc, zeros_acc)
    h1, g2 = _mid(h0, g1, S1, dis, b1.reshape(1, D), W2)
    S2 = _scatter_kernel(g2, rc, zeros_acc)
    h2, g3 = _mid(h1, g2, S2, dis, b2.reshape(1, D), W3)
    S3 = _scatter_kernel(g3, rc, zeros_acc)
    h3, np_out, gp, _ = _fin(h2, g3, S3, dis, b3.reshape(1, D),
                             Wn1, bn1.reshape(1, D), Wn2, bn2.reshape(1, 13),
                             Wg1, bg1.reshape(1, D // 2), Wg2,
                             bg2.reshape(1, 1))

    return (h3[:N], np_out[:N, :10], np_out[:N, 10:13], gp)


# speedup vs baseline: 1.1445x; 1.0203x over previous
import functools

import jax
import jax.numpy as jnp
from jax import lax
from jax.experimental import pallas as pl
from jax.experimental.pallas import tpu as pltpu
from jax.experimental.pallas import tpu_sc as plsc

N = 100000
E = 1600000
FEAT = 16
D = 64

NC = 2
NT = 16
LANES = 16

CHUNK = 25088
NDUMP = 256
ACC = CHUNK + NDUMP
NPAD = 4 * CHUNK

BLK = 1024
NBLK = 98
NROWS = NBLK * BLK

EB = 128
E_PAD = ((E + NT * EB - 1) // (NT * EB)) * (NT * EB)
EPT = E_PAD // NT
NB = EPT // EB
E_EXT = E_PAD + 5 * EB
EPT2 = E_PAD // (NT * NC)
NB2 = EPT2 // EB

_mesh = plsc.VectorSubcoreMesh(core_axis_name="c", subcore_axis_name="s",
                               num_cores=NC, num_subcores=NT)


@functools.partial(
    pl.kernel,
    out_type=jax.ShapeDtypeStruct((NC, NROWS), jnp.float32),
    mesh=_mesh,
    scratch_types=[
        pltpu.VMEM((4, EB), jnp.int32),
        pltpu.VMEM((EB,), jnp.float32),
        pltpu.VMEM_SHARED((NROWS,), jnp.float32),
        pltpu.SemaphoreType.DMA,
        pltpu.SemaphoreType.DMA,
    ],
)
def _deg_kernel(col_h, zeros_h, degp_h, col_v, ones_v, acc_sh,
                sem_c, sem_s):
    c = lax.axis_index("c")
    s = lax.axis_index("s")
    for k in range(EB // LANES):
        ones_v[pl.ds(k * LANES, LANES)] = jnp.ones((LANES,), jnp.float32)
    DS = NROWS // NT
    pltpu.sync_copy(zeros_h.at[pl.ds(s * DS, DS)], acc_sh.at[pl.ds(s * DS, DS)])
    plsc.subcore_barrier()
    ebase = (c * NT + s) * EPT2

    def fire_col(b, slot):
        pltpu.async_copy(col_h.at[pl.ds(ebase + b * EB, EB)], col_v.at[slot],
                         sem_c)

    def wait_col():
        pltpu.make_async_copy(col_h.at[pl.ds(0, EB)], col_v.at[0], sem_c).wait()

    def wait_sc():
        pltpu.make_async_copy(ones_v, acc_sh.at[col_v.at[0]], sem_s).wait()

    for b in range(2):
        fire_col(b, b)

    def body(b, carry):
        fire_col(b + 2, lax.rem(b + 2, 4))
        wait_col()

        @pl.when(b >= 2)
        def _():
            wait_sc()

        pltpu.async_copy(ones_v, acc_sh.at[col_v.at[lax.rem(b, 4)]], sem_s,
                         add=True)
        return carry

    lax.fori_loop(0, NB2, body, 0)
    for _ in range(2):
        wait_sc()
        wait_col()
    plsc.subcore_barrier()
    pltpu.sync_copy(acc_sh.at[pl.ds(s * DS, DS)], degp_h.at[c, pl.ds(s * DS, DS)])


NS_R = 3
NS_I = 4
IDX_AHEAD = 2
GATH_AHEAD = 1
SC_BEHIND = 2


@functools.partial(
    pl.kernel,
    out_type=jax.ShapeDtypeStruct((NPAD, D), jnp.float32),
    mesh=_mesh,
    compiler_params=pltpu.CompilerParams(use_tc_tiling_on_sc=False),
    scratch_types=[
        pltpu.VMEM((NS_I, EB), jnp.int32),
        pltpu.VMEM((NS_I, EB), jnp.int32),
        pltpu.VMEM((NS_I, EB), jnp.int32),
        pltpu.VMEM((NS_R, EB, D), jnp.float32),
        pltpu.VMEM_SHARED((ACC, D), jnp.float32),
        pltpu.SemaphoreType.DMA,
        pltpu.SemaphoreType.DMA,
        pltpu.SemaphoreType.DMA,
        pltpu.SemaphoreType.DMA,
    ],
)
def _scatter_kernel(g_h, row_h, col_h, zeros_h, out_h,
                    row_v, col_v, lcol_v, rows_v, acc_sh,
                    sem_r, sem_c, sem_g, sem_s):
    c = lax.axis_index("c")
    s = lax.axis_index("s")
    ebase = s * EPT
    ZS = ACC // NT
    WS = CHUNK // NT

    def fire_idx(b, slot):
        off = ebase + b * EB
        pltpu.async_copy(row_h.at[pl.ds(off, EB)], row_v.at[slot], sem_r)
        pltpu.async_copy(col_h.at[pl.ds(off, EB)], col_v.at[slot], sem_c)

    def wait_idx():
        pltpu.make_async_copy(row_h.at[pl.ds(0, EB)], row_v.at[0], sem_r).wait()
        pltpu.make_async_copy(col_h.at[pl.ds(0, EB)], col_v.at[0], sem_c).wait()

    def fire_gather(bslot, rslot):
        pltpu.async_copy(g_h.at[row_v.at[bslot]], rows_v.at[rslot], sem_g)

    def wait_gather():
        pltpu.make_async_copy(g_h.at[row_v.at[0]], rows_v.at[0], sem_g).wait()

    def fire_scatter(bslot, rslot):
        pltpu.async_copy(rows_v.at[rslot], acc_sh.at[lcol_v.at[bslot]], sem_s,
                         add=True)

    def wait_scatter():
        pltpu.make_async_copy(rows_v.at[0], acc_sh.at[lcol_v.at[0]],
                              sem_s).wait()

    for p in range(2):
        lo = (c * 2 + p) * CHUNK
        pltpu.sync_copy(zeros_h.at[pl.ds(s * ZS, ZS)], acc_sh.at[pl.ds(s * ZS, ZS)])
        plsc.subcore_barrier()

        for b in range(IDX_AHEAD):
            fire_idx(b, b)
        for b in range(GATH_AHEAD):
            wait_idx()
            fire_gather(b, b)

        def body(b, carry):
            fire_idx(b + IDX_AHEAD, lax.rem(b + IDX_AHEAD, NS_I))
            wait_idx()

            @pl.when(b >= SC_BEHIND)
            def _():
                wait_scatter()

            fire_gather(lax.rem(b + GATH_AHEAD, NS_I),
                        lax.rem(b + GATH_AHEAD, NS_R))
            kb = lax.rem(b, NS_I)
            for k in range(EB // LANES):
                cv = col_v[kb, pl.ds(k * LANES, LANES)]
                l = cv - lo
                valid = (l >= 0) & (l < CHUNK)
                dump = CHUNK + (cv & (NDUMP - 1))
                lcol_v[kb, pl.ds(k * LANES, LANES)] = jnp.where(valid, l, dump)
            wait_gather()
            fire_scatter(kb, lax.rem(b, NS_R))
            return carry

        lax.fori_loop(0, NB, body, 0)
        for _ in range(SC_BEHIND):
            wait_scatter()
        for _ in range(GATH_AHEAD):
            wait_gather()
        for _ in range(IDX_AHEAD - GATH_AHEAD):
            wait_idx()
        plsc.subcore_barrier()
        pltpu.sync_copy(acc_sh.at[pl.ds(s * WS, WS)],
                        out_h.at[pl.ds(lo + s * WS, WS)])
        plsc.subcore_barrier()


def _tc0_body(x_ref, degp_ref, We_ref, be_ref, W1_ref, h_ref, g_ref, dis_ref):
    deg = degp_ref[0] + degp_ref[1] + 1.0
    dis = lax.rsqrt(deg)
    h = jnp.dot(x_ref[...], We_ref[...], preferred_element_type=jnp.float32) + be_ref[...]
    g = jnp.dot(h, W1_ref[...], preferred_element_type=jnp.float32) * dis
    h_ref[...] = h
    g_ref[...] = g
    dis_ref[...] = dis


def _tc0(x_pad, degp3, W_emb, b_emb2, W1):
    return pl.pallas_call(
        _tc0_body,
        grid=(NBLK,),
        in_specs=[
            pl.BlockSpec((BLK, FEAT), lambda i: (i, 0)),
            pl.BlockSpec((NC, BLK, 1), lambda i: (0, i, 0)),
            pl.BlockSpec((FEAT, D), lambda i: (0, 0)),
            pl.BlockSpec((1, D), lambda i: (0, 0)),
            pl.BlockSpec((D, D), lambda i: (0, 0)),
        ],
        out_specs=[
            pl.BlockSpec((BLK, D), lambda i: (i, 0)),
            pl.BlockSpec((BLK, D), lambda i: (i, 0)),
            pl.BlockSpec((BLK, 1), lambda i: (i, 0)),
        ],
        out_shape=[
            jax.ShapeDtypeStruct((NROWS, D), jnp.float32),
            jax.ShapeDtypeStruct((NROWS, D), jnp.float32),
            jax.ShapeDtypeStruct((NROWS, 1), jnp.float32),
        ],
    )(x_pad, degp3, W_emb, b_emb2, W1)


def _mid_body(h_ref, g_ref, S_ref, dis_ref, b_ref, W_ref, hn_ref, gn_ref):
    dis = dis_ref[...]
    hn = h_ref[...] + jnp.maximum(dis * (S_ref[...] + g_ref[...]) + b_ref[...], 0.0)
    gn = jnp.dot(hn, W_ref[...], preferred_element_type=jnp.float32) * dis
    hn_ref[...] = hn
    gn_ref[...] = gn


def _mid(h, g, S, dis, b2, W_next):
    return pl.pallas_call(
        _mid_body,
        grid=(NBLK,),
        in_specs=[
            pl.BlockSpec((BLK, D), lambda i: (i, 0)),
            pl.BlockSpec((BLK, D), lambda i: (i, 0)),
            pl.BlockSpec((BLK, D), lambda i: (i, 0)),
            pl.BlockSpec((BLK, 1), lambda i: (i, 0)),
            pl.BlockSpec((1, D), lambda i: (0, 0)),
            pl.BlockSpec((D, D), lambda i: (0, 0)),
        ],
        out_specs=[
            pl.BlockSpec((BLK, D), lambda i: (i, 0)),
            pl.BlockSpec((BLK, D), lambda i: (i, 0)),
        ],
        out_shape=[
            jax.ShapeDtypeStruct((NROWS, D), jnp.float32),
            jax.ShapeDtypeStruct((NROWS, D), jnp.float32),
        ],
    )(h, g, S, dis, b2, W_next)


def _fin_body(h_ref, g_ref, S_ref, dis_ref, b_ref, Wn1_ref, bn1_ref, Wn2_ref,
              bn2_ref, Wg1_ref, bg1_ref, Wg2_ref, bg2_ref,
              h_out_ref, np_ref, gp_ref, hsum_ref):
    i = pl.program_id(0)
    h = h_ref[...] + jnp.maximum(
        dis_ref[...] * (S_ref[...] + g_ref[...]) + b_ref[...], 0.0)
    t = jnp.maximum(
        jnp.dot(h, Wn1_ref[...], preferred_element_type=jnp.float32) + bn1_ref[...], 0.0)
    np_ref[...] = jnp.dot(t, Wn2_ref[...], preferred_element_type=jnp.float32) + bn2_ref[...]
    h_out_ref[...] = h
    ridx = i * BLK + lax.broadcasted_iota(jnp.int32, (BLK, 1), 0)
    hm = jnp.where(ridx < N, h, 0.0)

    @pl.when(i == 0)
    def _():
        hsum_ref[...] = jnp.zeros_like(hsum_ref)

    hsum_ref[...] += jnp.sum(hm, axis=0, keepdims=True)

    @pl.when(i == NBLK - 1)
    def _():
        gm = hsum_ref[...] * (1.0 / N)
        tg = jnp.maximum(
            jnp.dot(gm, Wg1_ref[...], preferred_element_type=jnp.float32)
            + bg1_ref[...], 0.0)
        z = jnp.dot(tg, Wg2_ref[...], preferred_element_type=jnp.float32) + bg2_ref[...]
        gp_ref[...] = 1.0 / (1.0 + jnp.exp(-z))


def _fin(h, g, S, dis, b2, Wn1, bn1_2, Wn2, bn2_2, Wg1, bg1_2, Wg2, bg2_2):
    return pl.pallas_call(
        _fin_body,
        grid=(NBLK,),
        in_specs=[
            pl.BlockSpec((BLK, D), lambda i: (i, 0)),
            pl.BlockSpec((BLK, D), lambda i: (i, 0)),
            pl.BlockSpec((BLK, D), lambda i: (i, 0)),
            pl.BlockSpec((BLK, 1), lambda i: (i, 0)),
            pl.BlockSpec((1, D), lambda i: (0, 0)),
            pl.BlockSpec((D, D), lambda i: (0, 0)),
            pl.BlockSpec((1, D), lambda i: (0, 0)),
            pl.BlockSpec((D, 13), lambda i: (0, 0)),
            pl.BlockSpec((1, 13), lambda i: (0, 0)),
            pl.BlockSpec((D, D // 2), lambda i: (0, 0)),
            pl.BlockSpec((1, D // 2), lambda i: (0, 0)),
            pl.BlockSpec((D // 2, 1), lambda i: (0, 0)),
            pl.BlockSpec((1, 1), lambda i: (0, 0)),
        ],
        out_specs=[
            pl.BlockSpec((BLK, D), lambda i: (i, 0)),
            pl.BlockSpec((BLK, 13), lambda i: (i, 0)),
            pl.BlockSpec((1, 1), lambda i: (0, 0)),
            pl.BlockSpec((1, D), lambda i: (0, 0)),
        ],
        out_shape=[
            jax.ShapeDtypeStruct((NROWS, D), jnp.float32),
            jax.ShapeDtypeStruct((NROWS, 13), jnp.float32),
            jax.ShapeDtypeStruct((1, 1), jnp.float32),
            jax.ShapeDtypeStruct((1, D), jnp.float32),
        ],
    )(h, g, S, dis, b2, Wn1, bn1_2, Wn2, bn2_2, Wg1, bg1_2, Wg2, bg2_2)


def kernel(x, edge_index, W_emb, b_emb, W1, b1, W2, b2, W3, b3,
           Wn1, bn1, Wn2, bn2, Wg1, bg1, Wg2, bg2):
    padn = E_EXT - E
    j = jnp.arange(padn, dtype=jnp.int32)
    row_p = jnp.concatenate([edge_index[0], (j * 8191) % N])
    col_p = jnp.concatenate([edge_index[1], N + (j % 256)])

    x_pad = jnp.pad(x, ((0, NROWS - N), (0, 0)))
    zeros_acc = jnp.zeros((ACC, D), jnp.float32)
    zeros_deg = jnp.zeros((NROWS,), jnp.float32)

    degp = _deg_kernel(col_p, zeros_deg)
    degp3 = degp.reshape(NC, NROWS, 1)

    h0, g1, dis = _tc0(x_pad, degp3, W_emb, b_emb.reshape(1, D), W1)
    S1 = _scatter_kernel(g1, row_p, col_p, zeros_acc)
    h1, g2 = _mid(h0, g1, S1, dis, b1.reshape(1, D), W2)
    S2 = _scatter_kernel(g2, row_p, col_p, zeros_acc)
    h2, g3 = _mid(h1, g2, S2, dis, b2.reshape(1, D), W3)
    S3 = _scatter_kernel(g3, row_p, col_p, zeros_acc)
    h3, np_out, gp, _ = _fin(h2, g3, S3, dis, b3.reshape(1, D),
                             Wn1, bn1.reshape(1, D), Wn2, bn2.reshape(1, 13),
                             Wg1, bg1.reshape(1, D // 2), Wg2,
                             bg2.reshape(1, 1))

    return (h3[:N], np_out[:N, :10], np_out[:N, 10:13], gp)
